# Initial kernel scaffold; baseline (speedup 1.0000x reference)
import jax
import jax.numpy as jnp
from jax.experimental import pallas as pl


def kernel(z, emb_w):
    raise NotImplementedError("write your pallas kernel here")


# speedup vs baseline: 1.1380x; 1.1380x over previous
import jax
import jax.numpy as jnp
from jax.experimental import pallas as pl
from jax.experimental.pallas import tpu as pltpu

_NUM_EMB = 1024
_DIM = 64
_BETA = 0.25
_ROWS = 16384
_BLOCK = 512
_GRID = _ROWS // _BLOCK


def _vq_body(z_ref, w_ref, oh_ref, idx_ref, zq_ref, loss_ref):
    z_c = z_ref[0]
    w = w_ref[...]
    w2 = jnp.sum(w * w, axis=1)
    z2 = jnp.sum(z_c * z_c, axis=0)
    prod = jax.lax.dot_general(
        z_c, w, (((0,), (1,)), ((), ())),
        preferred_element_type=jnp.float32)
    obj = z2[:, None] + w2[None, :] - 2.0 * prod
    codes = jax.lax.broadcasted_iota(jnp.int32, (_BLOCK, _NUM_EMB), 1)
    m = jnp.min(obj, axis=1)
    idx = jnp.min(jnp.where(obj == m[:, None], codes, _NUM_EMB),
                  axis=1).astype(jnp.int32)
    oh = (codes == idx[:, None]).astype(jnp.float32)
    oh_ref[...] = oh
    idx_ref[...] = idx.reshape(1, 1, _BLOCK)

    zq_c = jax.lax.dot_general(
        w, oh, (((0,), (1,)), ((), ())),
        preferred_element_type=jnp.float32)
    zq_ref[...] = zq_c[None]

    diff = zq_c - z_c
    part = jnp.sum(diff * diff).reshape(1, 1)

    @pl.when(pl.program_id(0) == 0)
    def _init():
        loss_ref[...] = jnp.zeros((1, 1), jnp.float32)

    loss_ref[...] += part


def kernel(z, emb_w):
    z3 = z.reshape(16, 64, 1024)
    oh, idx, zq3, loss_sum = pl.pallas_call(
        _vq_body,
        grid=(_GRID,),
        in_specs=[
            pl.BlockSpec((1, 64, _BLOCK), lambda g: (g // 2, 0, g % 2)),
            pl.BlockSpec((_NUM_EMB, _DIM), lambda g: (0, 0)),
        ],
        out_specs=[
            pl.BlockSpec((_BLOCK, _NUM_EMB), lambda g: (g, 0)),
            pl.BlockSpec((1, 1, _BLOCK), lambda g: (g, 0, 0)),
            pl.BlockSpec((1, 64, _BLOCK), lambda g: (g // 2, 0, g % 2)),
            pl.BlockSpec((1, 1), lambda g: (0, 0)),
        ],
        out_shape=[
            jax.ShapeDtypeStruct((_ROWS, _NUM_EMB), jnp.float32),
            jax.ShapeDtypeStruct((_GRID, 1, _BLOCK), jnp.int32),
            jax.ShapeDtypeStruct((16, 64, 1024), jnp.float32),
            jax.ShapeDtypeStruct((1, 1), jnp.float32),
        ],
    )(z3, emb_w)
    loss = (1.0 + _BETA) * loss_sum[0, 0] / (_ROWS * _DIM)
    z_quantized = zq3.reshape(16, 64, 32, 32)
    return (loss, z_quantized, oh, idx.reshape(_ROWS))


# speedup vs baseline: 1.3204x; 1.1603x over previous
import jax
import jax.numpy as jnp
from jax.experimental import pallas as pl
from jax.experimental.pallas import tpu as pltpu

_NUM_EMB = 1024
_DIM = 64
_BETA = 0.25
_ROWS = 16384
_BLOCK = 1024
_GRID = _ROWS // _BLOCK


def _vq_body(z_ref, w_ref, oh_ref, idx_ref, zq_ref, loss_ref, w2_ref):
    w = w_ref[...]

    @pl.when(pl.program_id(0) == 0)
    def _init():
        w2_ref[...] = jnp.sum(w * w, axis=1).reshape(1, _NUM_EMB)
        loss_ref[...] = jnp.zeros((1, 1), jnp.float32)

    z_c = z_ref[0]
    z2 = jnp.sum(z_c * z_c, axis=0)
    prod = jax.lax.dot_general(
        z_c, w, (((0,), (1,)), ((), ())),
        preferred_element_type=jnp.float32)
    obj = z2[:, None] + w2_ref[...] - 2.0 * prod
    codes = jax.lax.broadcasted_iota(jnp.int32, (_BLOCK, _NUM_EMB), 1)
    m = jnp.min(obj, axis=1)
    idx = jnp.min(jnp.where(obj == m[:, None], codes, _NUM_EMB),
                  axis=1).astype(jnp.int32)
    oh = (codes == idx[:, None]).astype(jnp.float32)
    oh_ref[...] = oh
    idx_ref[...] = idx.reshape(1, 1, _BLOCK)

    zq_c = jax.lax.dot_general(
        w, oh, (((0,), (1,)), ((), ())),
        preferred_element_type=jnp.float32)
    zq_ref[...] = zq_c[None]

    loss_ref[...] += jnp.sum(m).reshape(1, 1)


def kernel(z, emb_w):
    z3 = z.reshape(16, 64, 1024)
    nblk = 1024 // _BLOCK if _BLOCK <= 1024 else 1
    if _BLOCK <= 1024:
        zmap = lambda g: (g // nblk, 0, g % nblk)
        zblk = (1, 64, _BLOCK)
    else:
        zmap = lambda g: (g, 0, 0)
        zblk = (_BLOCK // 1024, 64, 1024)
    oh, idx, zq3, loss_sum = pl.pallas_call(
        _vq_body,
        grid=(_GRID,),
        in_specs=[
            pl.BlockSpec(zblk, zmap),
            pl.BlockSpec((_NUM_EMB, _DIM), lambda g: (0, 0)),
        ],
        out_specs=[
            pl.BlockSpec((_BLOCK, _NUM_EMB), lambda g: (g, 0)),
            pl.BlockSpec((1, 1, _BLOCK), lambda g: (g, 0, 0)),
            pl.BlockSpec(zblk, zmap),
            pl.BlockSpec((1, 1), lambda g: (0, 0)),
        ],
        out_shape=[
            jax.ShapeDtypeStruct((_ROWS, _NUM_EMB), jnp.float32),
            jax.ShapeDtypeStruct((_GRID, 1, _BLOCK), jnp.int32),
            jax.ShapeDtypeStruct((16, 64, 1024), jnp.float32),
            jax.ShapeDtypeStruct((1, 1), jnp.float32),
        ],
        scratch_shapes=[pltpu.VMEM((1, _NUM_EMB), jnp.float32)],
    )(z3, emb_w)
    loss = (1.0 + _BETA) * loss_sum[0, 0] / (_ROWS * _DIM)
    z_quantized = zq3.reshape(16, 64, 32, 32)
    return (loss, z_quantized, oh, idx.reshape(_ROWS))
